# baseline (speedup 1.0000x reference)
import functools

import jax
import jax.numpy as jnp
from jax import lax
from jax.experimental import pallas as pl
from jax.experimental.pallas import tpu as pltpu
from jax.experimental.pallas import tpu_sc as plsc

B = 16384
P = 512
J = 4096
NM = 512
NJ = 1024
NL = 512

NC = 2
NS = 16
NW = NC * NS
L = 16

ROWS_PER_W = B // NW
R = 8
NG = ROWS_PER_W // R

CF_MMW = 0
CF_MMM = CF_MMW + NM
CF_MMH = CF_MMM + NM
CF_JW = CF_MMH + NM
CF_JM = CF_JW + NJ
CF_JH = CF_JM + NJ
CF_LS = CF_JH + NJ
CF_LO = CF_LS + NL
CF_LW = CF_LO + NL
CF_RMIN = CF_LW + NL
CF_RMAX = CF_RMIN + NL
CF_TOT = CF_RMAX + NL

CI_MM = 0
CI_J = CI_MM + NM
CI_LR = CI_J + NJ
CI_LT = CI_LR + NL
CI_TOT = CI_LT + NL


def _sc_partials(jp_flat, mp_flat, cf, ci):
    mesh = plsc.VectorSubcoreMesh(core_axis_name="c", subcore_axis_name="s")

    @functools.partial(
        pl.kernel,
        mesh=mesh,
        out_type=jax.ShapeDtypeStruct((NW * L,), jnp.float32),
        compiler_params=pltpu.CompilerParams(needs_layout_passes=False),
        scratch_types=[
            pltpu.VMEM((R * J,), jnp.float32),
            pltpu.VMEM((R * P,), jnp.float32),
            pltpu.VMEM((CF_TOT,), jnp.float32),
            pltpu.VMEM((CI_TOT,), jnp.int32),
            pltpu.VMEM((L,), jnp.float32),
        ],
    )
    def k(jp_hbm, mp_hbm, cf_hbm, ci_hbm, out_hbm, jbuf, mbuf, cfv, civ, accv):
        cid = lax.axis_index("c")
        sid = lax.axis_index("s")
        wid = sid * NC + cid
        base = wid * ROWS_PER_W

        pltpu.sync_copy(cf_hbm, cfv)
        pltpu.sync_copy(ci_hbm, civ)

        def group(g, acc):
            r0 = base + g * R
            pltpu.sync_copy(jp_hbm.at[pl.ds(r0 * J, R * J)], jbuf)
            pltpu.sync_copy(mp_hbm.at[pl.ds(r0 * P, R * P)], mbuf)

            def jchunk(c, acc):
                o = c * L
                idx = civ[pl.ds(CI_J + o, L)]
                w = cfv[pl.ds(CF_JW + o, L)]
                wm = cfv[pl.ds(CF_JM + o, L)]
                wh = cfv[pl.ds(CF_JH + o, L)]
                for r in range(R):
                    p = plsc.load_gather(jbuf, [idx + (r * J)])
                    v = jnp.maximum(jnp.abs(p * w - wm) - wh, 0.0)
                    acc = acc + v * v
                return acc

            acc = lax.fori_loop(0, NJ // L, jchunk, acc)

            def mchunk(c, acc):
                o = c * L
                idx = civ[pl.ds(CI_MM + o, L)]
                w = cfv[pl.ds(CF_MMW + o, L)]
                wm = cfv[pl.ds(CF_MMM + o, L)]
                wh = cfv[pl.ds(CF_MMH + o, L)]
                for r in range(R):
                    p = plsc.load_gather(mbuf, [idx + (r * P)])
                    v = jnp.maximum(jnp.abs(p * w - wm) - wh, 0.0)
                    acc = acc + v * v
                return acc

            acc = lax.fori_loop(0, NM // L, mchunk, acc)

            def lchunk(c, acc):
                o = c * L
                ridx = civ[pl.ds(CI_LR + o, L)]
                tidx = civ[pl.ds(CI_LT + o, L)]
                ws = cfv[pl.ds(CF_LS + o, L)]
                wo = cfv[pl.ds(CF_LO + o, L)]
                lw = cfv[pl.ds(CF_LW + o, L)]
                rmn = cfv[pl.ds(CF_RMIN + o, L)]
                rmx = cfv[pl.ds(CF_RMAX + o, L)]
                for r in range(R):
                    rf = plsc.load_gather(mbuf, [ridx + (r * P)])
                    tg = plsc.load_gather(mbuf, [tidx + (r * P)])
                    d = rf * ws - wo - tg * lw
                    act = (rf >= rmn) & (rf <= rmx)
                    dd = jnp.where(act, d, 0.0)
                    acc = acc + dd * dd
                return acc

            acc = lax.fori_loop(0, NL // L, lchunk, acc)
            return acc

        acc = lax.fori_loop(0, NG, group, jnp.zeros((L,), jnp.float32))
        accv[...] = acc
        pltpu.sync_copy(accv, out_hbm.at[pl.ds(wid * L, L)])

    return k(jp_flat, mp_flat, cf, ci)


def kernel(model_parameters, joint_parameters, minmax_min, minmax_max,
           minmax_weight, minmaxjoint_min, minmaxjoint_max,
           minmaxjoint_weight, linear_scale, linear_offset, linear_weight,
           linear_range_min, linear_range_max, minmax_parameter_index,
           minmaxjoint_index, linear_refidx, linear_targetidx):
    f32 = jnp.float32
    mm_w = minmax_weight.astype(f32)
    mm_wm = mm_w * 0.5 * (minmax_min + minmax_max)
    mm_wh = mm_w * 0.5 * (minmax_max - minmax_min)
    j_w = minmaxjoint_weight.astype(f32)
    j_wm = j_w * 0.5 * (minmaxjoint_min + minmaxjoint_max)
    j_wh = j_w * 0.5 * (minmaxjoint_max - minmaxjoint_min)
    l_ws = linear_weight * linear_scale
    l_wo = linear_weight * linear_offset
    l_w = linear_weight.astype(f32)

    cf = jnp.concatenate([
        mm_w, mm_wm, mm_wh,
        j_w, j_wm, j_wh,
        l_ws, l_wo, l_w,
        linear_range_min.astype(f32), linear_range_max.astype(f32),
    ]).astype(f32)
    ci = jnp.concatenate([
        minmax_parameter_index, minmaxjoint_index,
        linear_refidx, linear_targetidx,
    ]).astype(jnp.int32)

    partials = _sc_partials(
        joint_parameters.reshape(B * J),
        model_parameters.reshape(B * P),
        cf, ci,
    )
    return jnp.sum(partials)


# speedup vs baseline: 1.3997x; 1.3997x over previous
import functools

import jax
import jax.numpy as jnp
from jax import lax
from jax.experimental import pallas as pl
from jax.experimental.pallas import tpu as pltpu
from jax.experimental.pallas import tpu_sc as plsc

B = 16384
P = 512
J = 4096
NM = 512
NJ = 1024
NL = 512

NC = 2
NS = 16
NW = NC * NS
L = 16

ROWS_PER_W = B // NW
R = 8
NG = ROWS_PER_W // R

CF_MMW = 0
CF_MMM = CF_MMW + NM
CF_MMH = CF_MMM + NM
CF_JW = CF_MMH + NM
CF_JM = CF_JW + NJ
CF_JH = CF_JM + NJ
CF_LS = CF_JH + NJ
CF_LO = CF_LS + NL
CF_LW = CF_LO + NL
CF_RMIN = CF_LW + NL
CF_RMAX = CF_RMIN + NL
CF_TOT = CF_RMAX + NL

CI_MM = 0
CI_J = CI_MM + NM
CI_LR = CI_J + NJ
CI_LT = CI_LR + NL
CI_TOT = CI_LT + NL


def _sc_partials(jp_flat, mp_flat, cf, ci):
    mesh = plsc.VectorSubcoreMesh(core_axis_name="c", subcore_axis_name="s")

    @functools.partial(
        pl.kernel,
        mesh=mesh,
        out_type=jax.ShapeDtypeStruct((NW * L,), jnp.float32),
        compiler_params=pltpu.CompilerParams(needs_layout_passes=False),
        scratch_types=[
            pltpu.VMEM((2 * R * J,), jnp.float32),
            pltpu.VMEM((2 * R * P,), jnp.float32),
            pltpu.VMEM((CF_TOT,), jnp.float32),
            pltpu.VMEM((CI_TOT,), jnp.int32),
            pltpu.VMEM((L,), jnp.float32),
            pltpu.SemaphoreType.DMA,
            pltpu.SemaphoreType.DMA,
            pltpu.SemaphoreType.DMA,
            pltpu.SemaphoreType.DMA,
        ],
    )
    def k(jp_hbm, mp_hbm, cf_hbm, ci_hbm, out_hbm, jbuf, mbuf, cfv, civ, accv,
          sj0, sj1, sm0, sm1):
        cid = lax.axis_index("c")
        sid = lax.axis_index("s")
        wid = sid * NC + cid
        base = wid * ROWS_PER_W
        jsems = (sj0, sj1)
        msems = (sm0, sm1)

        pltpu.sync_copy(cf_hbm, cfv)
        pltpu.sync_copy(ci_hbm, civ)

        def start_slot(g, slot):
            r0 = base + g * R
            pltpu.async_copy(jp_hbm.at[pl.ds(r0 * J, R * J)],
                             jbuf.at[pl.ds(slot * R * J, R * J)], jsems[slot])
            pltpu.async_copy(mp_hbm.at[pl.ds(r0 * P, R * P)],
                             mbuf.at[pl.ds(slot * R * P, R * P)], msems[slot])

        def wait_slot(slot):
            pltpu.make_async_copy(
                jp_hbm.at[pl.ds(0, R * J)],
                jbuf.at[pl.ds(slot * R * J, R * J)], jsems[slot]).wait()
            pltpu.make_async_copy(
                mp_hbm.at[pl.ds(0, R * P)],
                mbuf.at[pl.ds(slot * R * P, R * P)], msems[slot]).wait()

        def compute(slot, acc):
            joff = slot * R * J
            moff = slot * R * P

            def jchunk(c, acc):
                o = c * L
                idx = civ[pl.ds(CI_J + o, L)]
                w = cfv[pl.ds(CF_JW + o, L)]
                wm = cfv[pl.ds(CF_JM + o, L)]
                wh = cfv[pl.ds(CF_JH + o, L)]
                for r in range(R):
                    p = plsc.load_gather(jbuf, [idx + (joff + r * J)])
                    v = jnp.maximum(jnp.abs(p * w - wm) - wh, 0.0)
                    acc = acc + v * v
                return acc

            acc = lax.fori_loop(0, NJ // L, jchunk, acc)

            def mchunk(c, acc):
                o = c * L
                idx = civ[pl.ds(CI_MM + o, L)]
                w = cfv[pl.ds(CF_MMW + o, L)]
                wm = cfv[pl.ds(CF_MMM + o, L)]
                wh = cfv[pl.ds(CF_MMH + o, L)]
                for r in range(R):
                    p = plsc.load_gather(mbuf, [idx + (moff + r * P)])
                    v = jnp.maximum(jnp.abs(p * w - wm) - wh, 0.0)
                    acc = acc + v * v
                return acc

            acc = lax.fori_loop(0, NM // L, mchunk, acc)

            def lchunk(c, acc):
                o = c * L
                ridx = civ[pl.ds(CI_LR + o, L)]
                tidx = civ[pl.ds(CI_LT + o, L)]
                ws = cfv[pl.ds(CF_LS + o, L)]
                wo = cfv[pl.ds(CF_LO + o, L)]
                lw = cfv[pl.ds(CF_LW + o, L)]
                rmn = cfv[pl.ds(CF_RMIN + o, L)]
                rmx = cfv[pl.ds(CF_RMAX + o, L)]
                for r in range(R):
                    rf = plsc.load_gather(mbuf, [ridx + (moff + r * P)])
                    tg = plsc.load_gather(mbuf, [tidx + (moff + r * P)])
                    d = rf * ws - wo - tg * lw
                    act = (rf >= rmn) & (rf <= rmx)
                    dd = jnp.where(act, d, 0.0)
                    acc = acc + dd * dd
                return acc

            acc = lax.fori_loop(0, NL // L, lchunk, acc)
            return acc

        start_slot(0, 0)

        def pair(i, acc):
            g = 2 * i
            start_slot(g + 1, 1)
            wait_slot(0)
            acc = compute(0, acc)

            @pl.when(g + 2 < NG)
            def _():
                start_slot(g + 2, 0)

            wait_slot(1)
            acc = compute(1, acc)
            return acc

        acc = lax.fori_loop(0, NG // 2, pair, jnp.zeros((L,), jnp.float32))
        accv[...] = acc
        pltpu.sync_copy(accv, out_hbm.at[pl.ds(wid * L, L)])

    return k(jp_flat, mp_flat, cf, ci)


def kernel(model_parameters, joint_parameters, minmax_min, minmax_max,
           minmax_weight, minmaxjoint_min, minmaxjoint_max,
           minmaxjoint_weight, linear_scale, linear_offset, linear_weight,
           linear_range_min, linear_range_max, minmax_parameter_index,
           minmaxjoint_index, linear_refidx, linear_targetidx):
    f32 = jnp.float32
    mm_w = minmax_weight.astype(f32)
    mm_wm = mm_w * 0.5 * (minmax_min + minmax_max)
    mm_wh = mm_w * 0.5 * (minmax_max - minmax_min)
    j_w = minmaxjoint_weight.astype(f32)
    j_wm = j_w * 0.5 * (minmaxjoint_min + minmaxjoint_max)
    j_wh = j_w * 0.5 * (minmaxjoint_max - minmaxjoint_min)
    l_ws = linear_weight * linear_scale
    l_wo = linear_weight * linear_offset
    l_w = linear_weight.astype(f32)

    cf = jnp.concatenate([
        mm_w, mm_wm, mm_wh,
        j_w, j_wm, j_wh,
        l_ws, l_wo, l_w,
        linear_range_min.astype(f32), linear_range_max.astype(f32),
    ]).astype(f32)
    ci = jnp.concatenate([
        minmax_parameter_index, minmaxjoint_index,
        linear_refidx, linear_targetidx,
    ]).astype(jnp.int32)

    partials = _sc_partials(
        joint_parameters.reshape(B * J),
        model_parameters.reshape(B * P),
        cf, ci,
    )
    return jnp.sum(partials)
